# baseline (speedup 1.0000x reference)
import functools

import jax
import jax.numpy as jnp
from jax import lax
from jax.experimental import pallas as pl
from jax.experimental.pallas import tpu as pltpu
from jax.experimental.pallas import tpu_sc as plsc

B = 4
S = 4096
D = 768
LANES = 16
NUM_CORES = 2
NUM_SUBCORES = 16
NW = NUM_CORES * NUM_SUBCORES
SPW = S // NW
RPW = B * SPW
SB = 16
C = B * SB
NCH = SPW // SB
PR = 2 * SB
NSLOT = 2
VECS_PER_ROW = D // LANES


def _body(x_hbm, table_hbm, pos_hbm, out_hbm, idx_v, pos_v, rows_v,
          gsem, osem, psem):
    cid = lax.axis_index("c")
    sid = lax.axis_index("s")
    wid = sid * NUM_CORES + cid
    s0 = wid * SPW

    pltpu.sync_copy(x_hbm.at[pl.ds(wid * RPW, RPW)], idx_v)

    gdesc = [None] * NSLOT
    odesc = [None] * NSLOT

    def issue_gather(t):
        slot = t % NSLOT
        if odesc[slot] is not None:
            for d in odesc[slot]:
                d.wait()
        gdesc[slot] = pltpu.async_copy(
            table_hbm.at[idx_v.at[pl.ds(t * C, C)]], rows_v.at[slot],
            gsem.at[slot])

    pdesc = pltpu.async_copy(pos_hbm.at[pl.ds(s0, PR)], pos_v, psem)
    issue_gather(0)

    for t in range(NCH):
        cur = t % NSLOT
        if t + 1 < NCH:
            issue_gather(t + 1)
        gdesc[cur].wait()
        if t % 2 == 0:
            pdesc.wait()
        poff = (t % 2) * SB

        def add_srow(sr, carry, cur=cur, poff=poff):
            for j in range(VECS_PER_ROW):
                sl = pl.ds(j * LANES, LANES)
                v = pos_v[poff + sr, sl]
                for bb in range(B):
                    plsc.addupdate(rows_v.at[cur, bb * SB + sr, sl], v)
            return carry

        lax.fori_loop(0, SB, add_srow, 0)

        odesc[cur] = [
            pltpu.async_copy(
                rows_v.at[cur, pl.ds(bb * SB, SB)],
                out_hbm.at[pl.ds(bb * S + s0 + t * SB, SB)], osem.at[cur])
            for bb in range(B)
        ]
        if t % 2 == 1 and t + 1 < NCH:
            pdesc = pltpu.async_copy(
                pos_hbm.at[pl.ds(s0 + (t // 2 + 1) * PR, PR)], pos_v, psem)

    for slot in range(NSLOT):
        if odesc[slot] is not None:
            for d in odesc[slot]:
                d.wait()


@jax.jit
def _embed(x_perm, table, pos_table):
    mesh = plsc.VectorSubcoreMesh(core_axis_name="c", subcore_axis_name="s")
    kfn = functools.partial(
        pl.kernel,
        out_type=jax.ShapeDtypeStruct((B * S, D), jnp.float32),
        mesh=mesh,
        scratch_types=[
            pltpu.VMEM((RPW,), jnp.int32),
            pltpu.VMEM((PR, D), jnp.float32),
            pltpu.VMEM((NSLOT, C, D), jnp.float32),
            pltpu.SemaphoreType.DMA((NSLOT,)),
            pltpu.SemaphoreType.DMA((NSLOT,)),
            pltpu.SemaphoreType.DMA,
        ],
    )(_body)
    return kfn(x_perm, table, pos_table)


def kernel(x, table, pos_table):
    x_perm = (x.reshape(B, NW, NCH, SB).transpose(1, 2, 0, 3)
              .reshape(NW * RPW).astype(jnp.int32))
    out = _embed(x_perm, table, pos_table)
    return out.reshape(B, S, D)
